# baseline (speedup 1.0000x reference)
import dataclasses

import jax
import jax.numpy as jnp
from jax import lax
from jax.experimental import pallas as pl
from jax.experimental.pallas import tpu as pltpu
from jax.experimental.pallas import tpu_sc as plsc

B = 16384
D = 64
NC, NS, L = 2, 16, 16
NW = NC * NS
BPW = B // NW
CHUNK = 128
NCHUNK = BPW // CHUNK
NG = CHUNK // L
KD = D // L


def _body(h_hbm, r_hbm, t_hbm, nh_hbm, nt_hbm, et_hbm, rt_hbm,
          pos_hbm, neg_hbm,
          hi_v, ri_v, ti_v, nhi_v, nti_v,
          h_rows, r_rows, t_rows, nh_rows, nt_rows,
          p_pos, p_neg, pos_v, neg_v, sem):
    wid = lax.axis_index("s") * NC + lax.axis_index("c")
    base = wid * BPW

    pltpu.sync_copy(h_hbm.at[pl.ds(base, BPW)], hi_v)
    pltpu.sync_copy(r_hbm.at[pl.ds(base, BPW)], ri_v)
    pltpu.sync_copy(t_hbm.at[pl.ds(base, BPW)], ti_v)
    pltpu.sync_copy(nh_hbm.at[pl.ds(base, BPW)], nhi_v)
    pltpu.sync_copy(nt_hbm.at[pl.ds(base, BPW)], nti_v)

    jv = lax.iota(jnp.int32, L)

    @pl.loop(0, NCHUNK)
    def _chunk(ci):
        off = ci * CHUNK
        cps = [
            pltpu.async_copy(et_hbm.at[hi_v.at[pl.ds(off, CHUNK)]], h_rows, sem),
            pltpu.async_copy(rt_hbm.at[ri_v.at[pl.ds(off, CHUNK)]], r_rows, sem),
            pltpu.async_copy(et_hbm.at[ti_v.at[pl.ds(off, CHUNK)]], t_rows, sem),
            pltpu.async_copy(et_hbm.at[nhi_v.at[pl.ds(off, CHUNK)]], nh_rows, sem),
            pltpu.async_copy(et_hbm.at[nti_v.at[pl.ds(off, CHUNK)]], nt_rows, sem),
        ]
        for cp in cps:
            cp.wait()

        @pl.loop(0, NG)
        def _group(g):
            for j in range(L):
                row = g * L + j
                pp = None
                pn = None
                for k in range(KD):
                    sl = pl.ds(k * L, L)
                    hv = h_rows[row, sl]
                    rv = r_rows[row, sl]
                    tv = t_rows[row, sl]
                    nhv = nh_rows[row, sl]
                    ntv = nt_rows[row, sl]
                    ap = jnp.abs(hv + rv - tv)
                    an = jnp.abs(nhv + rv - ntv)
                    pp = ap if pp is None else pp + ap
                    pn = an if pn is None else pn + an
                p_pos[j, pl.ds(0, L)] = pp
                p_neg[j, pl.ds(0, L)] = pn
            sp = jnp.zeros((L,), jnp.float32)
            sn = jnp.zeros((L,), jnp.float32)
            for c in range(L):
                cv = jnp.full((L,), c, jnp.int32)
                sp = sp + plsc.load_gather(p_pos, [jv, cv])
                sn = sn + plsc.load_gather(p_neg, [jv, cv])
            pos_v[pl.ds(off + g * L, L)] = -sp
            neg_v[pl.ds(off + g * L, L)] = -sn

    pltpu.sync_copy(pos_v, pos_hbm.at[pl.ds(base, BPW)])
    pltpu.sync_copy(neg_v, neg_hbm.at[pl.ds(base, BPW)])


def kernel(h, r, t, neg_h, neg_t, entity_table, relation_table):
    mesh = plsc.VectorSubcoreMesh(core_axis_name="c", subcore_axis_name="s")
    out = jax.ShapeDtypeStruct((B,), jnp.float32)
    cp = pltpu.CompilerParams(needs_layout_passes=False, use_tc_tiling_on_sc=False)
    kfn = pl.kernel(
        _body,
        out_type=(out, out),
        mesh=mesh,
        compiler_params=cp,
        scratch_types=[
            pltpu.VMEM((BPW,), jnp.int32),
            pltpu.VMEM((BPW,), jnp.int32),
            pltpu.VMEM((BPW,), jnp.int32),
            pltpu.VMEM((BPW,), jnp.int32),
            pltpu.VMEM((BPW,), jnp.int32),
            pltpu.VMEM((CHUNK, D), jnp.float32),
            pltpu.VMEM((CHUNK, D), jnp.float32),
            pltpu.VMEM((CHUNK, D), jnp.float32),
            pltpu.VMEM((CHUNK, D), jnp.float32),
            pltpu.VMEM((CHUNK, D), jnp.float32),
            pltpu.VMEM((L, 17), jnp.float32),
            pltpu.VMEM((L, 17), jnp.float32),
            pltpu.VMEM((BPW,), jnp.float32),
            pltpu.VMEM((BPW,), jnp.float32),
            pltpu.SemaphoreType.DMA,
        ],
    )
    return kfn(h, r, t, neg_h, neg_t, entity_table, relation_table)


# speedup vs baseline: 1.0853x; 1.0853x over previous
import dataclasses

import jax
import jax.numpy as jnp
from jax import lax
from jax.experimental import pallas as pl
from jax.experimental.pallas import tpu as pltpu
from jax.experimental.pallas import tpu_sc as plsc

B = 16384
D = 64
DP = 128
NC, NS, L = 2, 16, 16
NW = NC * NS
BPW = B // NW
CHUNK = 128
NCHUNK = BPW // CHUNK
NG = CHUNK // L
KD = D // L


def _body(h_hbm, r_hbm, t_hbm, nh_hbm, nt_hbm, et_hbm, rt_hbm,
          pos_hbm, neg_hbm,
          hi_v, ri_v, ti_v, nhi_v, nti_v,
          h_rows, r_rows, t_rows, nh_rows, nt_rows,
          p_pos, p_neg, pos_v, neg_v, sem):
    wid = lax.axis_index("s") * NC + lax.axis_index("c")
    base = wid * BPW

    pltpu.sync_copy(h_hbm.at[pl.ds(base, BPW)], hi_v)
    pltpu.sync_copy(r_hbm.at[pl.ds(base, BPW)], ri_v)
    pltpu.sync_copy(t_hbm.at[pl.ds(base, BPW)], ti_v)
    pltpu.sync_copy(nh_hbm.at[pl.ds(base, BPW)], nhi_v)
    pltpu.sync_copy(nt_hbm.at[pl.ds(base, BPW)], nti_v)

    jv = lax.iota(jnp.int32, L)

    @pl.loop(0, NCHUNK)
    def _chunk(ci):
        off = ci * CHUNK
        cps = [
            pltpu.async_copy(et_hbm.at[hi_v.at[pl.ds(off, CHUNK)]], h_rows, sem),
            pltpu.async_copy(rt_hbm.at[ri_v.at[pl.ds(off, CHUNK)]], r_rows, sem),
            pltpu.async_copy(et_hbm.at[ti_v.at[pl.ds(off, CHUNK)]], t_rows, sem),
            pltpu.async_copy(et_hbm.at[nhi_v.at[pl.ds(off, CHUNK)]], nh_rows, sem),
            pltpu.async_copy(et_hbm.at[nti_v.at[pl.ds(off, CHUNK)]], nt_rows, sem),
        ]
        for cp in cps:
            cp.wait()

        @pl.loop(0, NG)
        def _group(g):
            for j in range(L):
                row = g * L + j
                pp = None
                pn = None
                for k in range(KD):
                    sl = pl.ds(k * L, L)
                    hv = h_rows[row, sl]
                    rv = r_rows[row, sl]
                    tv = t_rows[row, sl]
                    nhv = nh_rows[row, sl]
                    ntv = nt_rows[row, sl]
                    ap = jnp.abs(hv + rv - tv)
                    an = jnp.abs(nhv + rv - ntv)
                    pp = ap if pp is None else pp + ap
                    pn = an if pn is None else pn + an
                p_pos[j, pl.ds(0, L)] = pp
                p_neg[j, pl.ds(0, L)] = pn
            sp = jnp.zeros((L,), jnp.float32)
            sn = jnp.zeros((L,), jnp.float32)
            for c in range(L):
                cv = jnp.full((L,), c, jnp.int32)
                sp = sp + plsc.load_gather(p_pos, [jv, cv])
                sn = sn + plsc.load_gather(p_neg, [jv, cv])
            pos_v[pl.ds(off + g * L, L)] = -sp
            neg_v[pl.ds(off + g * L, L)] = -sn

    pltpu.sync_copy(pos_v, pos_hbm.at[pl.ds(base, BPW)])
    pltpu.sync_copy(neg_v, neg_hbm.at[pl.ds(base, BPW)])


def kernel(h, r, t, neg_h, neg_t, entity_table, relation_table):
    et_pad = jnp.pad(entity_table, ((0, 0), (0, DP - D)))
    rt_pad = jnp.pad(relation_table, ((0, 0), (0, DP - D)))
    mesh = plsc.VectorSubcoreMesh(core_axis_name="c", subcore_axis_name="s")
    out = jax.ShapeDtypeStruct((B,), jnp.float32)
    cp = pltpu.CompilerParams(needs_layout_passes=False)
    kfn = pl.kernel(
        _body,
        out_type=(out, out),
        mesh=mesh,
        compiler_params=cp,
        scratch_types=[
            pltpu.VMEM((BPW,), jnp.int32),
            pltpu.VMEM((BPW,), jnp.int32),
            pltpu.VMEM((BPW,), jnp.int32),
            pltpu.VMEM((BPW,), jnp.int32),
            pltpu.VMEM((BPW,), jnp.int32),
            pltpu.VMEM((CHUNK, DP), jnp.float32),
            pltpu.VMEM((CHUNK, DP), jnp.float32),
            pltpu.VMEM((CHUNK, DP), jnp.float32),
            pltpu.VMEM((CHUNK, DP), jnp.float32),
            pltpu.VMEM((CHUNK, DP), jnp.float32),
            pltpu.VMEM((L, 17), jnp.float32),
            pltpu.VMEM((L, 17), jnp.float32),
            pltpu.VMEM((BPW,), jnp.float32),
            pltpu.VMEM((BPW,), jnp.float32),
            pltpu.SemaphoreType.DMA,
        ],
    )
    return kfn(h, r, t, neg_h, neg_t, et_pad, rt_pad)


# speedup vs baseline: 1.2286x; 1.1320x over previous
import jax
import jax.numpy as jnp
from jax import lax
from jax.experimental import pallas as pl
from jax.experimental.pallas import tpu as pltpu
from jax.experimental.pallas import tpu_sc as plsc

B = 16384
D = 64
DP = 128
NE = 1000000
NC, NS, L = 2, 16, 16
NW = NC * NS
BPW = B // NW
CHUNK = 128
NCH = BPW // CHUNK
NG = CHUNK // L
KD = D // L

CONV_BE = 2048
CONV_HB = CONV_BE // 2
CONV_GRID = -(-NE // CONV_BE)
NEP = CONV_GRID * CONV_HB


def _conv_body(x_ref, o_ref):
    x = x_ref[...]
    o_ref[:, 0:D] = x[:, 0:CONV_HB].T
    o_ref[:, D:DP] = x[:, CONV_HB:CONV_BE].T


def _convert(etT):
    return pl.pallas_call(
        _conv_body,
        grid=(CONV_GRID,),
        in_specs=[pl.BlockSpec((D, CONV_BE), lambda i: (0, i))],
        out_specs=pl.BlockSpec((CONV_HB, DP), lambda i: (i, 0)),
        out_shape=jax.ShapeDtypeStruct((NEP, DP), jnp.float32),
    )(etT)


def _score_body(h_hbm, r_hbm, t_hbm, nh_hbm, nt_hbm, et_hbm, rt_hbm,
                pos_hbm, neg_hbm,
                hi, ri, ti, nhi, nti,
                ph, pt, pnh, pnt,
                h_rows, r_rows, t_rows, nh_rows, nt_rows,
                p_pos, p_neg, pos_v, neg_v, sem):
    wid = lax.axis_index("s") * NC + lax.axis_index("c")
    base = wid * BPW

    for c in range(NCH):
        src = pl.ds(base + c * CHUNK, CHUNK)
        pltpu.sync_copy(h_hbm.at[src], hi.at[c])
        pltpu.sync_copy(r_hbm.at[src], ri.at[c])
        pltpu.sync_copy(t_hbm.at[src], ti.at[c])
        pltpu.sync_copy(nh_hbm.at[src], nhi.at[c])
        pltpu.sync_copy(nt_hbm.at[src], nti.at[c])

    one = jnp.ones((L,), jnp.int32)
    m10 = jnp.full((L,), CONV_HB - 1, jnp.int32)
    for c in range(NCH):
        for v in range(CHUNK // L):
            sl = pl.ds(v * L, L)
            for idx_ref, par_ref in ((hi, ph), (ti, pt), (nhi, pnh), (nti, pnt)):
                val = idx_ref[c, sl]
                par_ref[c, sl] = (val >> 10) & one
                idx_ref[c, sl] = ((val >> 11) << 10) | (val & m10)

    jv = lax.iota(jnp.int32, L)

    @pl.loop(0, NCH)
    def _chunk(ci):
        cps = [
            pltpu.async_copy(et_hbm.at[hi.at[ci]], h_rows, sem),
            pltpu.async_copy(rt_hbm.at[ri.at[ci]], r_rows, sem),
            pltpu.async_copy(et_hbm.at[ti.at[ci]], t_rows, sem),
            pltpu.async_copy(et_hbm.at[nhi.at[ci]], nh_rows, sem),
            pltpu.async_copy(et_hbm.at[nti.at[ci]], nt_rows, sem),
        ]
        for cp in cps:
            cp.wait()

        @pl.loop(0, NG)
        def _group(g):
            for j in range(L):
                row = g * L + j
                rowv = jnp.full((L,), row, jnp.int32)
                mh = plsc.load_gather(ph.at[ci], [rowv]) > 0
                mt = plsc.load_gather(pt.at[ci], [rowv]) > 0
                mnh = plsc.load_gather(pnh.at[ci], [rowv]) > 0
                mnt = plsc.load_gather(pnt.at[ci], [rowv]) > 0
                pp = None
                pn = None
                for k in range(KD):
                    lo = pl.ds(k * L, L)
                    hi_sl = pl.ds(D + k * L, L)
                    hv = jnp.where(mh, h_rows[row, hi_sl], h_rows[row, lo])
                    tv = jnp.where(mt, t_rows[row, hi_sl], t_rows[row, lo])
                    nhv = jnp.where(mnh, nh_rows[row, hi_sl], nh_rows[row, lo])
                    ntv = jnp.where(mnt, nt_rows[row, hi_sl], nt_rows[row, lo])
                    rv = r_rows[row, lo]
                    ap = jnp.abs(hv + rv - tv)
                    an = jnp.abs(nhv + rv - ntv)
                    pp = ap if pp is None else pp + ap
                    pn = an if pn is None else pn + an
                p_pos[j, pl.ds(0, L)] = pp
                p_neg[j, pl.ds(0, L)] = pn
            sp = jnp.zeros((L,), jnp.float32)
            sn = jnp.zeros((L,), jnp.float32)
            for c in range(L):
                cv = jnp.full((L,), c, jnp.int32)
                sp = sp + plsc.load_gather(p_pos, [jv, cv])
                sn = sn + plsc.load_gather(p_neg, [jv, cv])
            pos_v[pl.ds(ci * CHUNK + g * L, L)] = -sp
            neg_v[pl.ds(ci * CHUNK + g * L, L)] = -sn

    pltpu.sync_copy(pos_v, pos_hbm.at[pl.ds(base, BPW)])
    pltpu.sync_copy(neg_v, neg_hbm.at[pl.ds(base, BPW)])


def kernel(h, r, t, neg_h, neg_t, entity_table, relation_table):
    et2 = _convert(entity_table.T)
    rt_pad = jnp.pad(relation_table, ((0, 0), (0, DP - D)))
    mesh = plsc.VectorSubcoreMesh(core_axis_name="c", subcore_axis_name="s")
    out = jax.ShapeDtypeStruct((B,), jnp.float32)
    cp = pltpu.CompilerParams(needs_layout_passes=False)
    idx = pltpu.VMEM((NCH, CHUNK), jnp.int32)
    rows = pltpu.VMEM((CHUNK, DP), jnp.float32)
    kfn = pl.kernel(
        _score_body,
        out_type=(out, out),
        mesh=mesh,
        compiler_params=cp,
        scratch_types=[idx] * 9 + [rows] * 5 + [
            pltpu.VMEM((L, 17), jnp.float32),
            pltpu.VMEM((L, 17), jnp.float32),
            pltpu.VMEM((BPW,), jnp.float32),
            pltpu.VMEM((BPW,), jnp.float32),
            pltpu.SemaphoreType.DMA,
        ],
    )
    return kfn(h, r, t, neg_h, neg_t, et2, rt_pad)


# speedup vs baseline: 1.9297x; 1.5706x over previous
import jax
import jax.numpy as jnp
from jax import lax
from jax.experimental import pallas as pl
from jax.experimental.pallas import tpu as pltpu
from jax.experimental.pallas import tpu_sc as plsc

B = 16384
D = 64
DP = 128
NE = 1000000
NC, NS, L = 2, 16, 16
NW = NC * NS
BPW = B // NW
CHUNK = 128
NCH = BPW // CHUNK
NG = CHUNK // L
KD = D // L

CONV_BE = 8192
CONV_HB = CONV_BE // 2
CONV_GRID = -(-NE // CONV_BE)
NEP = CONV_GRID * CONV_HB


def _conv_body(x_ref, o_ref):
    x = x_ref[...]
    o_ref[...] = jnp.concatenate(
        [x[:, 0:CONV_HB].T, x[:, CONV_HB:CONV_BE].T], axis=1)


def _convert(etT):
    return pl.pallas_call(
        _conv_body,
        grid=(CONV_GRID,),
        in_specs=[pl.BlockSpec((D, CONV_BE), lambda i: (0, i))],
        out_specs=pl.BlockSpec((CONV_HB, DP), lambda i: (i, 0)),
        out_shape=jax.ShapeDtypeStruct((NEP, DP), jnp.float32),
    )(etT)


def _score_body(h_hbm, r_hbm, t_hbm, nh_hbm, nt_hbm, et_hbm, rt_hbm,
                pos_hbm, neg_hbm,
                hi, ri, ti, nhi, nti,
                ph, pt, pnh, pnt,
                h_rows, r_rows, t_rows, nh_rows, nt_rows,
                p_pos, p_neg, pos_v, neg_v, sem):
    wid = lax.axis_index("s") * NC + lax.axis_index("c")
    base = wid * BPW

    for c in range(NCH):
        src = pl.ds(base + c * CHUNK, CHUNK)
        pltpu.sync_copy(h_hbm.at[src], hi.at[c])
        pltpu.sync_copy(r_hbm.at[src], ri.at[c])
        pltpu.sync_copy(t_hbm.at[src], ti.at[c])
        pltpu.sync_copy(nh_hbm.at[src], nhi.at[c])
        pltpu.sync_copy(nt_hbm.at[src], nti.at[c])

    hb_bits = CONV_HB.bit_length() - 1
    one = jnp.ones((L,), jnp.int32)
    mlow = jnp.full((L,), CONV_HB - 1, jnp.int32)
    for c in range(NCH):
        for v in range(CHUNK // L):
            sl = pl.ds(v * L, L)
            for idx_ref, par_ref in ((hi, ph), (ti, pt), (nhi, pnh), (nti, pnt)):
                val = idx_ref[c, sl]
                par_ref[c, sl] = (val >> hb_bits) & one
                idx_ref[c, sl] = ((val >> (hb_bits + 1)) << hb_bits) | (val & mlow)

    jv = lax.iota(jnp.int32, L)

    @pl.loop(0, NCH)
    def _chunk(ci):
        cps = [
            pltpu.async_copy(et_hbm.at[hi.at[ci]], h_rows, sem),
            pltpu.async_copy(rt_hbm.at[ri.at[ci]], r_rows, sem),
            pltpu.async_copy(et_hbm.at[ti.at[ci]], t_rows, sem),
            pltpu.async_copy(et_hbm.at[nhi.at[ci]], nh_rows, sem),
            pltpu.async_copy(et_hbm.at[nti.at[ci]], nt_rows, sem),
        ]
        for cp in cps:
            cp.wait()

        @pl.loop(0, NG)
        def _group(g):
            for j in range(L):
                row = g * L + j
                rowv = jnp.full((L,), row, jnp.int32)
                mh = plsc.load_gather(ph.at[ci], [rowv]) > 0
                mt = plsc.load_gather(pt.at[ci], [rowv]) > 0
                mnh = plsc.load_gather(pnh.at[ci], [rowv]) > 0
                mnt = plsc.load_gather(pnt.at[ci], [rowv]) > 0
                pp = None
                pn = None
                for k in range(KD):
                    lo = pl.ds(k * L, L)
                    hi_sl = pl.ds(D + k * L, L)
                    hv = jnp.where(mh, h_rows[row, hi_sl], h_rows[row, lo])
                    tv = jnp.where(mt, t_rows[row, hi_sl], t_rows[row, lo])
                    nhv = jnp.where(mnh, nh_rows[row, hi_sl], nh_rows[row, lo])
                    ntv = jnp.where(mnt, nt_rows[row, hi_sl], nt_rows[row, lo])
                    rv = r_rows[row, lo]
                    ap = jnp.abs(hv + rv - tv)
                    an = jnp.abs(nhv + rv - ntv)
                    pp = ap if pp is None else pp + ap
                    pn = an if pn is None else pn + an
                p_pos[j, pl.ds(0, L)] = pp
                p_neg[j, pl.ds(0, L)] = pn
            sp = jnp.zeros((L,), jnp.float32)
            sn = jnp.zeros((L,), jnp.float32)
            for c in range(L):
                cv = jnp.full((L,), c, jnp.int32)
                sp = sp + plsc.load_gather(p_pos, [jv, cv])
                sn = sn + plsc.load_gather(p_neg, [jv, cv])
            pos_v[pl.ds(ci * CHUNK + g * L, L)] = -sp
            neg_v[pl.ds(ci * CHUNK + g * L, L)] = -sn

    pltpu.sync_copy(pos_v, pos_hbm.at[pl.ds(base, BPW)])
    pltpu.sync_copy(neg_v, neg_hbm.at[pl.ds(base, BPW)])


def kernel(h, r, t, neg_h, neg_t, entity_table, relation_table):
    et2 = _convert(entity_table.T)
    rt_pad = jnp.pad(relation_table, ((0, 0), (0, DP - D)))
    mesh = plsc.VectorSubcoreMesh(core_axis_name="c", subcore_axis_name="s")
    out = jax.ShapeDtypeStruct((B,), jnp.float32)
    cp = pltpu.CompilerParams(needs_layout_passes=False)
    idx = pltpu.VMEM((NCH, CHUNK), jnp.int32)
    rows = pltpu.VMEM((CHUNK, DP), jnp.float32)
    kfn = pl.kernel(
        _score_body,
        out_type=(out, out),
        mesh=mesh,
        compiler_params=cp,
        scratch_types=[idx] * 9 + [rows] * 5 + [
            pltpu.VMEM((L, 17), jnp.float32),
            pltpu.VMEM((L, 17), jnp.float32),
            pltpu.VMEM((BPW,), jnp.float32),
            pltpu.VMEM((BPW,), jnp.float32),
            pltpu.SemaphoreType.DMA,
        ],
    )
    return kfn(h, r, t, neg_h, neg_t, et2, rt_pad)


# speedup vs baseline: 2.3839x; 1.2353x over previous
import jax
import jax.numpy as jnp
from jax import lax
from jax.experimental import pallas as pl
from jax.experimental.pallas import tpu as pltpu
from jax.experimental.pallas import tpu_sc as plsc

B = 16384
D = 64
DP = 128
NE = 1000000
NC, NS, L = 2, 16, 16
NW = NC * NS
BPW = B // NW
CHUNK = 128
NCH = BPW // CHUNK
NG = CHUNK // L
KD = D // L

CONV_BE = 8192
CONV_HB = CONV_BE // 2
CONV_GRID = -(-NE // CONV_BE)
NEP = CONV_GRID * CONV_HB


def _conv_body(x_ref, o_ref):
    x = x_ref[...]
    o_ref[...] = jnp.concatenate([x[:, 0:CONV_HB], x[:, CONV_HB:CONV_BE]],
                                 axis=0).T


def _convert(etT):
    return pl.pallas_call(
        _conv_body,
        grid=(CONV_GRID,),
        in_specs=[pl.BlockSpec((D, CONV_BE), lambda i: (0, i))],
        out_specs=pl.BlockSpec((CONV_HB, DP), lambda i: (i, 0)),
        out_shape=jax.ShapeDtypeStruct((NEP, DP), jnp.float32),
    )(etT)


def _score_body(h_hbm, r_hbm, t_hbm, nh_hbm, nt_hbm, et_hbm, rt_hbm,
                pos_hbm, neg_hbm,
                hi, ri, ti, nhi, nti,
                ph, pt, pnh, pnt,
                h_rows, r_rows, t_rows, nh_rows, nt_rows,
                p_pos, p_neg, pos_v, neg_v, sem):
    wid = lax.axis_index("s") * NC + lax.axis_index("c")
    base = wid * BPW

    for c in range(NCH):
        src = pl.ds(base + c * CHUNK, CHUNK)
        pltpu.sync_copy(h_hbm.at[src], hi.at[c])
        pltpu.sync_copy(r_hbm.at[src], ri.at[c])
        pltpu.sync_copy(t_hbm.at[src], ti.at[c])
        pltpu.sync_copy(nh_hbm.at[src], nhi.at[c])
        pltpu.sync_copy(nt_hbm.at[src], nti.at[c])

    hb_bits = CONV_HB.bit_length() - 1
    one = jnp.ones((L,), jnp.int32)
    mlow = jnp.full((L,), CONV_HB - 1, jnp.int32)
    for c in range(NCH):
        for v in range(CHUNK // L):
            sl = pl.ds(v * L, L)
            for idx_ref, par_ref in ((hi, ph), (ti, pt), (nhi, pnh), (nti, pnt)):
                val = idx_ref[c, sl]
                par_ref[c, sl] = (val >> hb_bits) & one
                idx_ref[c, sl] = ((val >> (hb_bits + 1)) << hb_bits) | (val & mlow)

    jv = lax.iota(jnp.int32, L)

    @pl.loop(0, NCH)
    def _chunk(ci):
        cps = [
            pltpu.async_copy(et_hbm.at[hi.at[ci]], h_rows, sem),
            pltpu.async_copy(rt_hbm.at[ri.at[ci]], r_rows, sem),
            pltpu.async_copy(et_hbm.at[ti.at[ci]], t_rows, sem),
            pltpu.async_copy(et_hbm.at[nhi.at[ci]], nh_rows, sem),
            pltpu.async_copy(et_hbm.at[nti.at[ci]], nt_rows, sem),
        ]
        for cp in cps:
            cp.wait()

        @pl.loop(0, NG)
        def _group(g):
            for j in range(L):
                row = g * L + j
                rowv = jnp.full((L,), row, jnp.int32)
                mh = plsc.load_gather(ph.at[ci], [rowv]) > 0
                mt = plsc.load_gather(pt.at[ci], [rowv]) > 0
                mnh = plsc.load_gather(pnh.at[ci], [rowv]) > 0
                mnt = plsc.load_gather(pnt.at[ci], [rowv]) > 0
                pp = None
                pn = None
                for k in range(KD):
                    lo = pl.ds(k * L, L)
                    hi_sl = pl.ds(D + k * L, L)
                    hv = jnp.where(mh, h_rows[row, hi_sl], h_rows[row, lo])
                    tv = jnp.where(mt, t_rows[row, hi_sl], t_rows[row, lo])
                    nhv = jnp.where(mnh, nh_rows[row, hi_sl], nh_rows[row, lo])
                    ntv = jnp.where(mnt, nt_rows[row, hi_sl], nt_rows[row, lo])
                    rv = r_rows[row, lo]
                    ap = jnp.abs(hv + rv - tv)
                    an = jnp.abs(nhv + rv - ntv)
                    pp = ap if pp is None else pp + ap
                    pn = an if pn is None else pn + an
                p_pos[j, pl.ds(0, L)] = pp
                p_neg[j, pl.ds(0, L)] = pn
            sp = jnp.zeros((L,), jnp.float32)
            sn = jnp.zeros((L,), jnp.float32)
            for c in range(L):
                cv = jnp.full((L,), c, jnp.int32)
                sp = sp + plsc.load_gather(p_pos, [jv, cv])
                sn = sn + plsc.load_gather(p_neg, [jv, cv])
            pos_v[pl.ds(ci * CHUNK + g * L, L)] = -sp
            neg_v[pl.ds(ci * CHUNK + g * L, L)] = -sn

    pltpu.sync_copy(pos_v, pos_hbm.at[pl.ds(base, BPW)])
    pltpu.sync_copy(neg_v, neg_hbm.at[pl.ds(base, BPW)])


def kernel(h, r, t, neg_h, neg_t, entity_table, relation_table):
    et2 = _convert(entity_table.T)
    rt_pad = jnp.pad(relation_table, ((0, 0), (0, DP - D)))
    mesh = plsc.VectorSubcoreMesh(core_axis_name="c", subcore_axis_name="s")
    out = jax.ShapeDtypeStruct((B,), jnp.float32)
    cp = pltpu.CompilerParams(needs_layout_passes=False)
    idx = pltpu.VMEM((NCH, CHUNK), jnp.int32)
    rows = pltpu.VMEM((CHUNK, DP), jnp.float32)
    kfn = pl.kernel(
        _score_body,
        out_type=(out, out),
        mesh=mesh,
        compiler_params=cp,
        scratch_types=[idx] * 9 + [rows] * 5 + [
            pltpu.VMEM((L, 17), jnp.float32),
            pltpu.VMEM((L, 17), jnp.float32),
            pltpu.VMEM((BPW,), jnp.float32),
            pltpu.VMEM((BPW,), jnp.float32),
            pltpu.SemaphoreType.DMA,
        ],
    )
    return kfn(h, r, t, neg_h, neg_t, et2, rt_pad)


# speedup vs baseline: 2.6620x; 1.1167x over previous
import jax
import jax.numpy as jnp
from jax import lax
from jax.experimental import pallas as pl
from jax.experimental.pallas import tpu as pltpu
from jax.experimental.pallas import tpu_sc as plsc

B = 16384
D = 64
DP = 128
NE = 1000000
NC, NS, L = 2, 16, 16
NW = NC * NS
BPW = B // NW
CHUNK = 128
NCH = BPW // CHUNK
NG = CHUNK // L
KD = D // L

CONV_BE = 16384
CONV_HB = CONV_BE // 2
CONV_GRID = -(-NE // CONV_BE)
NEP = CONV_GRID * CONV_HB


def _conv_body(x_ref, o_ref):
    x = x_ref[...]
    o_ref[...] = jnp.concatenate([x[:, 0:CONV_HB], x[:, CONV_HB:CONV_BE]],
                                 axis=0).T


def _convert(etT):
    return pl.pallas_call(
        _conv_body,
        grid=(CONV_GRID,),
        in_specs=[pl.BlockSpec((D, CONV_BE), lambda i: (0, i))],
        out_specs=pl.BlockSpec((CONV_HB, DP), lambda i: (i, 0)),
        out_shape=jax.ShapeDtypeStruct((NEP, DP), jnp.float32),
    )(etT)


def _score_body(h_hbm, r_hbm, t_hbm, nh_hbm, nt_hbm, et_hbm, rt_hbm,
                pos_hbm, neg_hbm,
                hi, ri, ti, nhi, nti,
                ph, pt, pnh, pnt,
                h_rows, r_rows, t_rows, nh_rows, nt_rows,
                p_pos, p_neg, pos_v, neg_v, sem):
    wid = lax.axis_index("s") * NC + lax.axis_index("c")
    base = wid * BPW

    for c in range(NCH):
        src = pl.ds(base + c * CHUNK, CHUNK)
        pltpu.sync_copy(h_hbm.at[src], hi.at[c])
        pltpu.sync_copy(r_hbm.at[src], ri.at[c])
        pltpu.sync_copy(t_hbm.at[src], ti.at[c])
        pltpu.sync_copy(nh_hbm.at[src], nhi.at[c])
        pltpu.sync_copy(nt_hbm.at[src], nti.at[c])

    hb_bits = CONV_HB.bit_length() - 1
    one = jnp.ones((L,), jnp.int32)
    mlow = jnp.full((L,), CONV_HB - 1, jnp.int32)
    for c in range(NCH):
        for v in range(CHUNK // L):
            sl = pl.ds(v * L, L)
            for idx_ref, par_ref in ((hi, ph), (ti, pt), (nhi, pnh), (nti, pnt)):
                val = idx_ref[c, sl]
                par_ref[c, sl] = (val >> hb_bits) & one
                idx_ref[c, sl] = ((val >> (hb_bits + 1)) << hb_bits) | (val & mlow)

    jv = lax.iota(jnp.int32, L)

    @pl.loop(0, NCH)
    def _chunk(ci):
        cps = [
            pltpu.async_copy(et_hbm.at[hi.at[ci]], h_rows, sem),
            pltpu.async_copy(rt_hbm.at[ri.at[ci]], r_rows, sem),
            pltpu.async_copy(et_hbm.at[ti.at[ci]], t_rows, sem),
            pltpu.async_copy(et_hbm.at[nhi.at[ci]], nh_rows, sem),
            pltpu.async_copy(et_hbm.at[nti.at[ci]], nt_rows, sem),
        ]
        for cp in cps:
            cp.wait()

        @pl.loop(0, NG)
        def _group(g):
            for j in range(L):
                row = g * L + j
                rowv = jnp.full((L,), row, jnp.int32)
                mh = plsc.load_gather(ph.at[ci], [rowv]) > 0
                mt = plsc.load_gather(pt.at[ci], [rowv]) > 0
                mnh = plsc.load_gather(pnh.at[ci], [rowv]) > 0
                mnt = plsc.load_gather(pnt.at[ci], [rowv]) > 0
                pp = None
                pn = None
                for k in range(KD):
                    lo = pl.ds(k * L, L)
                    hi_sl = pl.ds(D + k * L, L)
                    hv = jnp.where(mh, h_rows[row, hi_sl], h_rows[row, lo])
                    tv = jnp.where(mt, t_rows[row, hi_sl], t_rows[row, lo])
                    nhv = jnp.where(mnh, nh_rows[row, hi_sl], nh_rows[row, lo])
                    ntv = jnp.where(mnt, nt_rows[row, hi_sl], nt_rows[row, lo])
                    rv = r_rows[row, lo]
                    ap = jnp.abs(hv + rv - tv)
                    an = jnp.abs(nhv + rv - ntv)
                    pp = ap if pp is None else pp + ap
                    pn = an if pn is None else pn + an
                p_pos[j, pl.ds(0, L)] = pp
                p_neg[j, pl.ds(0, L)] = pn
            sp = jnp.zeros((L,), jnp.float32)
            sn = jnp.zeros((L,), jnp.float32)
            for c in range(L):
                cv = jnp.full((L,), c, jnp.int32)
                sp = sp + plsc.load_gather(p_pos, [jv, cv])
                sn = sn + plsc.load_gather(p_neg, [jv, cv])
            pos_v[pl.ds(ci * CHUNK + g * L, L)] = -sp
            neg_v[pl.ds(ci * CHUNK + g * L, L)] = -sn

    pltpu.sync_copy(pos_v, pos_hbm.at[pl.ds(base, BPW)])
    pltpu.sync_copy(neg_v, neg_hbm.at[pl.ds(base, BPW)])


def kernel(h, r, t, neg_h, neg_t, entity_table, relation_table):
    et2 = _convert(entity_table.T)
    rt_pad = jnp.pad(relation_table, ((0, 0), (0, DP - D)))
    mesh = plsc.VectorSubcoreMesh(core_axis_name="c", subcore_axis_name="s")
    out = jax.ShapeDtypeStruct((B,), jnp.float32)
    cp = pltpu.CompilerParams(needs_layout_passes=False)
    idx = pltpu.VMEM((NCH, CHUNK), jnp.int32)
    rows = pltpu.VMEM((CHUNK, DP), jnp.float32)
    kfn = pl.kernel(
        _score_body,
        out_type=(out, out),
        mesh=mesh,
        compiler_params=cp,
        scratch_types=[idx] * 9 + [rows] * 5 + [
            pltpu.VMEM((L, 17), jnp.float32),
            pltpu.VMEM((L, 17), jnp.float32),
            pltpu.VMEM((BPW,), jnp.float32),
            pltpu.VMEM((BPW,), jnp.float32),
            pltpu.SemaphoreType.DMA,
        ],
    )
    return kfn(h, r, t, neg_h, neg_t, et2, rt_pad)


# speedup vs baseline: 2.7121x; 1.0188x over previous
import jax
import jax.numpy as jnp
from jax import lax
from jax.experimental import pallas as pl
from jax.experimental.pallas import tpu as pltpu
from jax.experimental.pallas import tpu_sc as plsc

B = 16384
D = 64
DP = 128
NE = 1000000
NC, NS, L = 2, 16, 16
NW = NC * NS
BPW = B // NW
CHUNK = 128
NCH = BPW // CHUNK
NG = CHUNK // L
KD = D // L

CONV_BE = 32768
CONV_HB = CONV_BE // 2
CONV_GRID = -(-NE // CONV_BE)
NEP = CONV_GRID * CONV_HB


def _conv_body(x_ref, o_ref):
    x = x_ref[...]
    o_ref[...] = jnp.concatenate([x[:, 0:CONV_HB], x[:, CONV_HB:CONV_BE]],
                                 axis=0).T


def _convert(etT):
    return pl.pallas_call(
        _conv_body,
        grid=(CONV_GRID,),
        in_specs=[pl.BlockSpec((D, CONV_BE), lambda i: (0, i))],
        out_specs=pl.BlockSpec((CONV_HB, DP), lambda i: (i, 0)),
        out_shape=jax.ShapeDtypeStruct((NEP, DP), jnp.float32),
    )(etT)


def _score_body(h_hbm, r_hbm, t_hbm, nh_hbm, nt_hbm, et_hbm, rt_hbm,
                pos_hbm, neg_hbm,
                hi, ri, ti, nhi, nti,
                ph, pt, pnh, pnt,
                h_rows, r_rows, t_rows, nh_rows, nt_rows,
                p_pos, p_neg, pos_v, neg_v, sem):
    wid = lax.axis_index("s") * NC + lax.axis_index("c")
    base = wid * BPW

    for c in range(NCH):
        src = pl.ds(base + c * CHUNK, CHUNK)
        pltpu.sync_copy(h_hbm.at[src], hi.at[c])
        pltpu.sync_copy(r_hbm.at[src], ri.at[c])
        pltpu.sync_copy(t_hbm.at[src], ti.at[c])
        pltpu.sync_copy(nh_hbm.at[src], nhi.at[c])
        pltpu.sync_copy(nt_hbm.at[src], nti.at[c])

    hb_bits = CONV_HB.bit_length() - 1
    one = jnp.ones((L,), jnp.int32)
    mlow = jnp.full((L,), CONV_HB - 1, jnp.int32)
    for c in range(NCH):
        for v in range(CHUNK // L):
            sl = pl.ds(v * L, L)
            for idx_ref, par_ref in ((hi, ph), (ti, pt), (nhi, pnh), (nti, pnt)):
                val = idx_ref[c, sl]
                par_ref[c, sl] = (val >> hb_bits) & one
                idx_ref[c, sl] = ((val >> (hb_bits + 1)) << hb_bits) | (val & mlow)

    jv = lax.iota(jnp.int32, L)

    @pl.loop(0, NCH)
    def _chunk(ci):
        cps = [
            pltpu.async_copy(et_hbm.at[hi.at[ci]], h_rows, sem),
            pltpu.async_copy(rt_hbm.at[ri.at[ci]], r_rows, sem),
            pltpu.async_copy(et_hbm.at[ti.at[ci]], t_rows, sem),
            pltpu.async_copy(et_hbm.at[nhi.at[ci]], nh_rows, sem),
            pltpu.async_copy(et_hbm.at[nti.at[ci]], nt_rows, sem),
        ]
        for cp in cps:
            cp.wait()

        @pl.loop(0, NG)
        def _group(g):
            for j in range(L):
                row = g * L + j
                rowv = jnp.full((L,), row, jnp.int32)
                mh = plsc.load_gather(ph.at[ci], [rowv]) > 0
                mt = plsc.load_gather(pt.at[ci], [rowv]) > 0
                mnh = plsc.load_gather(pnh.at[ci], [rowv]) > 0
                mnt = plsc.load_gather(pnt.at[ci], [rowv]) > 0
                pp = None
                pn = None
                for k in range(KD):
                    lo = pl.ds(k * L, L)
                    hi_sl = pl.ds(D + k * L, L)
                    hv = jnp.where(mh, h_rows[row, hi_sl], h_rows[row, lo])
                    tv = jnp.where(mt, t_rows[row, hi_sl], t_rows[row, lo])
                    nhv = jnp.where(mnh, nh_rows[row, hi_sl], nh_rows[row, lo])
                    ntv = jnp.where(mnt, nt_rows[row, hi_sl], nt_rows[row, lo])
                    rv = r_rows[row, lo]
                    ap = jnp.abs(hv + rv - tv)
                    an = jnp.abs(nhv + rv - ntv)
                    pp = ap if pp is None else pp + ap
                    pn = an if pn is None else pn + an
                p_pos[j, pl.ds(0, L)] = pp
                p_neg[j, pl.ds(0, L)] = pn
            sp = jnp.zeros((L,), jnp.float32)
            sn = jnp.zeros((L,), jnp.float32)
            for c in range(L):
                cv = jnp.full((L,), c, jnp.int32)
                sp = sp + plsc.load_gather(p_pos, [jv, cv])
                sn = sn + plsc.load_gather(p_neg, [jv, cv])
            pos_v[pl.ds(ci * CHUNK + g * L, L)] = -sp
            neg_v[pl.ds(ci * CHUNK + g * L, L)] = -sn

    pltpu.sync_copy(pos_v, pos_hbm.at[pl.ds(base, BPW)])
    pltpu.sync_copy(neg_v, neg_hbm.at[pl.ds(base, BPW)])


def kernel(h, r, t, neg_h, neg_t, entity_table, relation_table):
    et2 = _convert(entity_table.T)
    rt_pad = jnp.pad(relation_table, ((0, 0), (0, DP - D)))
    mesh = plsc.VectorSubcoreMesh(core_axis_name="c", subcore_axis_name="s")
    out = jax.ShapeDtypeStruct((B,), jnp.float32)
    cp = pltpu.CompilerParams(needs_layout_passes=False)
    idx = pltpu.VMEM((NCH, CHUNK), jnp.int32)
    rows = pltpu.VMEM((CHUNK, DP), jnp.float32)
    kfn = pl.kernel(
        _score_body,
        out_type=(out, out),
        mesh=mesh,
        compiler_params=cp,
        scratch_types=[idx] * 9 + [rows] * 5 + [
            pltpu.VMEM((L, 17), jnp.float32),
            pltpu.VMEM((L, 17), jnp.float32),
            pltpu.VMEM((BPW,), jnp.float32),
            pltpu.VMEM((BPW,), jnp.float32),
            pltpu.SemaphoreType.DMA,
        ],
    )
    return kfn(h, r, t, neg_h, neg_t, et2, rt_pad)


# speedup vs baseline: 2.7599x; 1.0176x over previous
import jax
import jax.numpy as jnp
from jax import lax
from jax.experimental import pallas as pl
from jax.experimental.pallas import tpu as pltpu
from jax.experimental.pallas import tpu_sc as plsc

B = 16384
D = 64
DP = 128
NE = 1000000
NC, NS, L = 2, 16, 16
NW = NC * NS
BPW = B // NW
CHUNK = 64
NCH = BPW // CHUNK
NG = CHUNK // L
KD = D // L

CONV_BE = 32768
CONV_HB = CONV_BE // 2
CONV_GRID = -(-NE // CONV_BE)
NEP = CONV_GRID * CONV_HB


def _conv_body(x_ref, o_ref):
    x = x_ref[...]
    o_ref[...] = jnp.concatenate([x[:, 0:CONV_HB], x[:, CONV_HB:CONV_BE]],
                                 axis=0).T


def _convert(etT):
    return pl.pallas_call(
        _conv_body,
        grid=(CONV_GRID,),
        in_specs=[pl.BlockSpec((D, CONV_BE), lambda i: (0, i))],
        out_specs=pl.BlockSpec((CONV_HB, DP), lambda i: (i, 0)),
        out_shape=jax.ShapeDtypeStruct((NEP, DP), jnp.float32),
    )(etT)


def _score_body(h_hbm, r_hbm, t_hbm, nh_hbm, nt_hbm, et_hbm, rt_hbm,
                pos_hbm, neg_hbm,
                hi, ri, ti, nhi, nti,
                ph, pt, pnh, pnt,
                h_ra, r_ra, t_ra, nh_ra, nt_ra,
                h_rb, r_rb, t_rb, nh_rb, nt_rb,
                p_pos, p_neg, pos_v, neg_v, sem):
    wid = lax.axis_index("s") * NC + lax.axis_index("c")
    base = wid * BPW

    for c in range(NCH):
        src = pl.ds(base + c * CHUNK, CHUNK)
        pltpu.sync_copy(h_hbm.at[src], hi.at[c])
        pltpu.sync_copy(r_hbm.at[src], ri.at[c])
        pltpu.sync_copy(t_hbm.at[src], ti.at[c])
        pltpu.sync_copy(nh_hbm.at[src], nhi.at[c])
        pltpu.sync_copy(nt_hbm.at[src], nti.at[c])

    hb_bits = CONV_HB.bit_length() - 1
    one = jnp.ones((L,), jnp.int32)
    mlow = jnp.full((L,), CONV_HB - 1, jnp.int32)
    for c in range(NCH):
        for v in range(CHUNK // L):
            sl = pl.ds(v * L, L)
            for idx_ref, par_ref in ((hi, ph), (ti, pt), (nhi, pnh), (nti, pnt)):
                val = idx_ref[c, sl]
                par_ref[c, sl] = (val >> hb_bits) & one
                idx_ref[c, sl] = ((val >> (hb_bits + 1)) << hb_bits) | (val & mlow)

    jv = lax.iota(jnp.int32, L)

    def fire(ci, h_r, r_r, t_r, nh_r, nt_r):
        pltpu.async_copy(et_hbm.at[hi.at[ci]], h_r, sem)
        pltpu.async_copy(rt_hbm.at[ri.at[ci]], r_r, sem)
        pltpu.async_copy(et_hbm.at[ti.at[ci]], t_r, sem)
        pltpu.async_copy(et_hbm.at[nhi.at[ci]], nh_r, sem)
        pltpu.async_copy(et_hbm.at[nti.at[ci]], nt_r, sem)

    def wait_like(ci, h_r, r_r, t_r, nh_r, nt_r):
        pltpu.make_async_copy(et_hbm.at[hi.at[ci]], h_r, sem).wait()
        pltpu.make_async_copy(rt_hbm.at[ri.at[ci]], r_r, sem).wait()
        pltpu.make_async_copy(et_hbm.at[ti.at[ci]], t_r, sem).wait()
        pltpu.make_async_copy(et_hbm.at[nhi.at[ci]], nh_r, sem).wait()
        pltpu.make_async_copy(et_hbm.at[nti.at[ci]], nt_r, sem).wait()

    def compute(ci, h_rows, r_rows, t_rows, nh_rows, nt_rows):
        @pl.loop(0, NG)
        def _group(g):
            for j in range(L):
                row = g * L + j
                rowv = jnp.full((L,), row, jnp.int32)
                mh = plsc.load_gather(ph.at[ci], [rowv]) > 0
                mt = plsc.load_gather(pt.at[ci], [rowv]) > 0
                mnh = plsc.load_gather(pnh.at[ci], [rowv]) > 0
                mnt = plsc.load_gather(pnt.at[ci], [rowv]) > 0
                pp = None
                pn = None
                for k in range(KD):
                    lo = pl.ds(k * L, L)
                    hi_sl = pl.ds(D + k * L, L)
                    hv = jnp.where(mh, h_rows[row, hi_sl], h_rows[row, lo])
                    tv = jnp.where(mt, t_rows[row, hi_sl], t_rows[row, lo])
                    nhv = jnp.where(mnh, nh_rows[row, hi_sl], nh_rows[row, lo])
                    ntv = jnp.where(mnt, nt_rows[row, hi_sl], nt_rows[row, lo])
                    rv = r_rows[row, lo]
                    ap = jnp.abs(hv + rv - tv)
                    an = jnp.abs(nhv + rv - ntv)
                    pp = ap if pp is None else pp + ap
                    pn = an if pn is None else pn + an
                p_pos[j, pl.ds(0, L)] = pp
                p_neg[j, pl.ds(0, L)] = pn
            sp = jnp.zeros((L,), jnp.float32)
            sn = jnp.zeros((L,), jnp.float32)
            for c in range(L):
                cv = jnp.full((L,), c, jnp.int32)
                sp = sp + plsc.load_gather(p_pos, [jv, cv])
                sn = sn + plsc.load_gather(p_neg, [jv, cv])
            pos_v[pl.ds(ci * CHUNK + g * L, L)] = -sp
            neg_v[pl.ds(ci * CHUNK + g * L, L)] = -sn

    fire(0, h_ra, r_ra, t_ra, nh_ra, nt_ra)

    @pl.loop(0, NCH // 2)
    def _pair(i):
        c0 = 2 * i
        wait_like(c0, h_ra, r_ra, t_ra, nh_ra, nt_ra)
        fire(c0 + 1, h_rb, r_rb, t_rb, nh_rb, nt_rb)
        compute(c0, h_ra, r_ra, t_ra, nh_ra, nt_ra)
        wait_like(c0 + 1, h_rb, r_rb, t_rb, nh_rb, nt_rb)

        @pl.when(c0 + 2 < NCH)
        def _():
            fire(c0 + 2, h_ra, r_ra, t_ra, nh_ra, nt_ra)

        compute(c0 + 1, h_rb, r_rb, t_rb, nh_rb, nt_rb)

    pltpu.sync_copy(pos_v, pos_hbm.at[pl.ds(base, BPW)])
    pltpu.sync_copy(neg_v, neg_hbm.at[pl.ds(base, BPW)])


def kernel(h, r, t, neg_h, neg_t, entity_table, relation_table):
    et2 = _convert(entity_table.T)
    rt_pad = jnp.pad(relation_table, ((0, 0), (0, DP - D)))
    mesh = plsc.VectorSubcoreMesh(core_axis_name="c", subcore_axis_name="s")
    out = jax.ShapeDtypeStruct((B,), jnp.float32)
    cp = pltpu.CompilerParams(needs_layout_passes=False)
    idx = pltpu.VMEM((NCH, CHUNK), jnp.int32)
    rows = pltpu.VMEM((CHUNK, DP), jnp.float32)
    kfn = pl.kernel(
        _score_body,
        out_type=(out, out),
        mesh=mesh,
        compiler_params=cp,
        scratch_types=[idx] * 9 + [rows] * 10 + [
            pltpu.VMEM((L, 17), jnp.float32),
            pltpu.VMEM((L, 17), jnp.float32),
            pltpu.VMEM((BPW,), jnp.float32),
            pltpu.VMEM((BPW,), jnp.float32),
            pltpu.SemaphoreType.DMA,
        ],
    )
    return kfn(h, r, t, neg_h, neg_t, et2, rt_pad)
